# baseline (speedup 1.0000x reference)
import jax
import jax.numpy as jnp
from jax import lax
from jax.experimental import pallas as pl
from jax.experimental.pallas import tpu as pltpu

K = 8
CH = 128
RL = 100


def _pos_body(t_ref, s_ref, o_ref):
    t = t_ref[...]
    s = s_ref[...]
    o_ref[...] = t[:, None, :] + s[None, :, :]


def _add_body(pos_ref, f_hbm, o_hbm, in_buf, out_buf, in_sem, out_sem):
    def start_in(chunk, slot):
        pltpu.make_async_copy(f_hbm.at[chunk], in_buf.at[slot],
                              in_sem.at[slot]).start()

    for k in range(K):
        start_in(k, k)

    def step(i, carry):
        slot = lax.rem(i, K)
        pltpu.make_async_copy(f_hbm.at[i], in_buf.at[slot],
                              in_sem.at[slot]).wait()

        @pl.when(i >= K)
        def _():
            pltpu.make_async_copy(out_buf.at[slot], o_hbm.at[i - K],
                                  out_sem.at[slot]).wait()

        out_buf[slot] = in_buf[slot] + pos_ref[lax.rem(i, 2)]

        pltpu.make_async_copy(out_buf.at[slot], o_hbm.at[i],
                              out_sem.at[slot]).start()

        @pl.when(i + K < CH)
        def _():
            start_in(i + K, slot)

        return carry

    lax.fori_loop(0, CH, step, 0)

    for k in range(K):
        pltpu.make_async_copy(out_buf.at[k], o_hbm.at[CH - K + k],
                              out_sem.at[k]).wait()


def kernel(feat, spatial_emb, temporal_emb):
    B, L, O, D = feat.shape
    OD = O * D

    pos = pl.pallas_call(
        _pos_body,
        out_shape=jax.ShapeDtypeStruct((L, O, D), feat.dtype),
    )(temporal_emb, spatial_emb)

    pos3 = pos.reshape(L // RL, RL, OD)
    feat3 = feat.reshape(CH, RL, OD)
    out = pl.pallas_call(
        _add_body,
        in_specs=[
            pl.BlockSpec((L // RL, RL, OD), lambda: (0, 0, 0)),
            pl.BlockSpec(memory_space=pl.ANY),
        ],
        out_specs=pl.BlockSpec(memory_space=pl.ANY),
        out_shape=jax.ShapeDtypeStruct((CH, RL, OD), feat.dtype),
        scratch_shapes=[
            pltpu.VMEM((K, RL, OD), jnp.float32),
            pltpu.VMEM((K, RL, OD), jnp.float32),
            pltpu.SemaphoreType.DMA((K,)),
            pltpu.SemaphoreType.DMA((K,)),
        ],
    )(pos3, feat3)
    return out.reshape(B, L, O, D)


# speedup vs baseline: 1.9325x; 1.9325x over previous
import jax
import jax.numpy as jnp
from jax import lax
from jax.experimental import pallas as pl
from jax.experimental.pallas import tpu as pltpu

K = 8
RL = 100


def _add_body(t_ref, s_ref, f_hbm, o_hbm, pos_v, in_buf, out_buf,
              in_sem, out_sem):
    B, L = f_hbm.shape[0], f_hbm.shape[1]
    CH = B * (L // RL)

    t = t_ref[...]
    s = s_ref[...]
    pos_v[...] = t[:, None, :] + s[None, :, :]

    def start_in(chunk, slot):
        b = lax.div(chunk, 2)
        l0 = lax.rem(chunk, 2) * RL
        pltpu.make_async_copy(f_hbm.at[b, pl.ds(l0, RL)], in_buf.at[slot],
                              in_sem.at[slot]).start()

    for k in range(K):
        start_in(k, k)

    def step(i, carry):
        slot = lax.rem(i, K)
        b = lax.div(i, 2)
        l0 = lax.rem(i, 2) * RL
        pltpu.make_async_copy(f_hbm.at[b, pl.ds(l0, RL)], in_buf.at[slot],
                              in_sem.at[slot]).wait()

        @pl.when(i >= K)
        def _():
            j = i - K
            pltpu.make_async_copy(
                out_buf.at[slot],
                o_hbm.at[lax.div(j, 2), pl.ds(lax.rem(j, 2) * RL, RL)],
                out_sem.at[slot]).wait()

        out_buf[slot] = in_buf[slot] + pos_v[pl.ds(l0, RL)]

        pltpu.make_async_copy(out_buf.at[slot],
                              o_hbm.at[b, pl.ds(l0, RL)],
                              out_sem.at[slot]).start()

        @pl.when(i + K < CH)
        def _():
            start_in(i + K, slot)

        return carry

    lax.fori_loop(0, CH, step, 0)

    for k in range(K):
        j = CH - K + k
        pltpu.make_async_copy(
            out_buf.at[k],
            o_hbm.at[lax.div(j, 2), pl.ds(lax.rem(j, 2) * RL, RL)],
            out_sem.at[k]).wait()


def kernel(feat, spatial_emb, temporal_emb):
    B, L, O, D = feat.shape
    return pl.pallas_call(
        _add_body,
        in_specs=[
            pl.BlockSpec((L, D), lambda: (0, 0)),
            pl.BlockSpec((O, D), lambda: (0, 0)),
            pl.BlockSpec(memory_space=pl.ANY),
        ],
        out_specs=pl.BlockSpec(memory_space=pl.ANY),
        out_shape=jax.ShapeDtypeStruct((B, L, O, D), feat.dtype),
        scratch_shapes=[
            pltpu.VMEM((L, O, D), jnp.float32),
            pltpu.VMEM((K, RL, O, D), jnp.float32),
            pltpu.VMEM((K, RL, O, D), jnp.float32),
            pltpu.SemaphoreType.DMA((K,)),
            pltpu.SemaphoreType.DMA((K,)),
        ],
    )(temporal_emb, spatial_emb, feat)
